# baseline (speedup 1.0000x reference)
import functools

import jax
import jax.numpy as jnp
from jax import lax
from jax.experimental import pallas as pl
from jax.experimental.pallas import tpu as pltpu
from jax.experimental.pallas import tpu_sc as plsc

N = 10000
E = 320000
F = 128
DE = 16
L = 4
C = 10
G = 64
BN_EPS = 1e-5

NC = 2
NS = 16
NW = NC * NS
CH = 128
CW0 = 62
CW1 = 96
EPAD = NS * CH * (CW0 + CW1)
NACC = 10240
RPS = NACC // NS


def _zero_vmem_rows(buf, rows, width):
    z16 = jnp.zeros((16,), jnp.float32)

    def body(i, _):
        for j in range(width // 16):
            buf[i, pl.ds(j * 16, 16)] = z16
        return 0

    lax.fori_loop(0, rows, body, 0)


def _segsum_body(tab_hbm, src_hbm, dst_hbm, out_hbm, sA, dA, sB, dB,
                 rows0, rows1, acc, sem0, sem1, *, gather):
    cid = lax.axis_index("c")
    sid = lax.axis_index("s")
    cw = jnp.where(cid == 0, CW0, CW1)
    base = jnp.where(cid == 0, sid * CW0, NS * CW0 + sid * CW1)

    _zero_vmem_rows(rows0, CH, F)

    def zbody(i, _):
        pltpu.sync_copy(rows0, acc.at[pl.ds(sid * RPS + i * CH, CH)])
        return 0

    lax.fori_loop(0, RPS // CH, zbody, 0)
    plsc.subcore_barrier()

    def body(j, _):
        row = base + j
        pltpu.sync_copy(dst_hbm.at[row], dA)
        if gather:
            pltpu.sync_copy(src_hbm.at[row], sA)
            pltpu.async_copy(tab_hbm.at[sA.at[0]], rows0, sem0).wait()
        else:
            pltpu.sync_copy(tab_hbm.at[row], rows0)
        pltpu.sync_copy(rows0, acc.at[dA.at[0]], add=True)
        return 0

    lax.fori_loop(0, cw, body, 0)
    plsc.subcore_barrier()

    pltpu.sync_copy(acc.at[pl.ds(sid * RPS, RPS)],
                    out_hbm.at[cid, pl.ds(sid * RPS, RPS)])


@functools.cache
def _sc_kernels():
    mesh = plsc.VectorSubcoreMesh(core_axis_name="c", subcore_axis_name="s",
                                  num_cores=NC, num_subcores=NS)

    scratches = [
        pltpu.VMEM((1, CH), jnp.int32),
        pltpu.VMEM((1, CH), jnp.int32),
        pltpu.VMEM((1, CH), jnp.int32),
        pltpu.VMEM((1, CH), jnp.int32),
        pltpu.VMEM((CH, F), jnp.float32),
        pltpu.VMEM((CH, F), jnp.float32),
        pltpu.VMEM_SHARED((NACC, F), jnp.float32),
        pltpu.SemaphoreType.DMA,
        pltpu.SemaphoreType.DMA,
    ]

    @functools.partial(
        pl.kernel,
        out_type=jax.ShapeDtypeStruct((NC, NACC, F), jnp.float32),
        mesh=mesh,
        scratch_types=scratches,
    )
    def _p_segsum(x_hbm, src3d, dst3d, out_hbm, sA, dA, sB, dB, rows0,
                  rows1, acc, sem0, sem1):
        _segsum_body(x_hbm, src3d, dst3d, out_hbm, sA, dA, sB, dB, rows0,
                     rows1, acc, sem0, sem1, gather=True)

    @functools.partial(
        pl.kernel,
        out_type=jax.ShapeDtypeStruct((NC, NACC, F), jnp.float32),
        mesh=mesh,
        scratch_types=scratches,
    )
    def _s_segsum(ea3d, src3d, dst3d, out_hbm, sA, dA, sB, dB, rows0,
                  rows1, acc, sem0, sem1):
        _segsum_body(ea3d, src3d, dst3d, out_hbm, sA, dA, sB, dB, rows0,
                     rows1, acc, sem0, sem1, gather=False)

    return _p_segsum, _s_segsum


def _embed_tc(h_ref, w_ref, b_ref, o_ref):
    o_ref[...] = jnp.dot(h_ref[...], w_ref[...],
                         preferred_element_type=jnp.float32) + b_ref[...]


def _layer_tc(x_ref, p_ref, s_ref, we_ref, wc_ref, bc_ref, g_ref, b_ref, o_ref):
    x = x_ref[...]
    s = s_ref[0, :N, :DE] + s_ref[1, :N, :DE]
    agg = p_ref[0, :N] + p_ref[1, :N] + jnp.dot(s, we_ref[...],
                                        preferred_element_type=jnp.float32)
    y = jnp.dot(x + agg, wc_ref[...],
                preferred_element_type=jnp.float32) + bc_ref[...]
    mu = jnp.mean(y, axis=0, keepdims=True)
    var = jnp.mean((y - mu) * (y - mu), axis=0, keepdims=True)
    y = (y - mu) * lax.rsqrt(var + BN_EPS)
    y = y * g_ref[...] + b_ref[...]
    o_ref[...] = jnp.maximum(y, 0.0) + x


def _pool_tc(x_ref, batch_ref, wout_ref, bout_ref, o_ref):
    x = x_ref[...]
    b = batch_ref[...]
    gids = lax.broadcasted_iota(jnp.int32, (G, 1), 0)
    onehot_t = (b == gids).astype(jnp.float32)
    sums = jnp.dot(onehot_t, x, preferred_element_type=jnp.float32)
    cnt = jnp.sum(onehot_t, axis=1, keepdims=True)
    pooled = sums / jnp.maximum(cnt, 1.0)
    o_ref[...] = jnp.dot(pooled, wout_ref[...],
                         preferred_element_type=jnp.float32) + bout_ref[...]


def kernel(h, edge_attr, edge_index, edge_feat_mat, pair_info, batch,
           W_embed, b_embed, Wconv, bconv, We, gamma, beta, Wout, bout):
    del edge_feat_mat, pair_info
    src = edge_index[0].astype(jnp.int32)
    dst = edge_index[1].astype(jnp.int32)
    pad = EPAD - E
    src3d = jnp.concatenate(
        [src, jnp.zeros((pad,), jnp.int32)]).reshape(NS * (CW0 + CW1), 1, CH)
    dst3d = jnp.concatenate(
        [dst, jnp.full((pad,), N, jnp.int32)]).reshape(NS * (CW0 + CW1), 1, CH)
    ea3d = jnp.pad(edge_attr, ((0, pad), (0, F - DE))).reshape(NS * (CW0 + CW1), CH, F)

    _p_segsum, _s_segsum = _sc_kernels()

    x = pl.pallas_call(
        _embed_tc,
        out_shape=jax.ShapeDtypeStruct((N, F), jnp.float32),
    )(h, W_embed, b_embed.reshape(1, F))

    s_part = _s_segsum(ea3d, src3d, dst3d)

    layer = pl.pallas_call(
        _layer_tc,
        out_shape=jax.ShapeDtypeStruct((N, F), jnp.float32),
    )
    for l in range(L):
        p_part = _p_segsum(x, src3d, dst3d)
        x = layer(x, p_part, s_part, We[l], Wconv[l], bconv[l].reshape(1, F),
                  gamma[l].reshape(1, F), beta[l].reshape(1, F))

    out = pl.pallas_call(
        _pool_tc,
        out_shape=jax.ShapeDtypeStruct((G, C), jnp.float32),
    )(x, batch.astype(jnp.int32).reshape(1, N), Wout, bout.reshape(1, C))
    return out


# speedup vs baseline: 1.1984x; 1.1984x over previous
import functools

import jax
import jax.numpy as jnp
from jax import lax
from jax.experimental import pallas as pl
from jax.experimental.pallas import tpu as pltpu
from jax.experimental.pallas import tpu_sc as plsc

N = 10000
E = 320000
F = 128
DE = 16
L = 4
C = 10
G = 64
BN_EPS = 1e-5

NC = 2
NS = 16
NW = NC * NS
CH = 128
CW0 = 96
CW1 = 62
EPAD = NS * CH * (CW0 + CW1)
NACC = 10240
RPS = NACC // NS


def _zero_vmem_rows(buf, rows, width):
    z16 = jnp.zeros((16,), jnp.float32)

    def body(i, _):
        for j in range(width // 16):
            buf[i, pl.ds(j * 16, 16)] = z16
        return 0

    lax.fori_loop(0, rows, body, 0)


def _segsum_body(tab_hbm, src_hbm, dst_hbm, out_hbm, sA, dA, sB, dB,
                 rows0, rows1, acc, sem0, sem1, *, gather):
    cid = lax.axis_index("c")
    sid = lax.axis_index("s")
    cw = jnp.where(cid == 0, CW0, CW1)
    base = jnp.where(cid == 0, sid * CW0, NS * CW0 + sid * CW1)

    _zero_vmem_rows(rows0, CH, F)

    def zbody(i, _):
        pltpu.sync_copy(rows0, acc.at[pl.ds(sid * RPS + i * CH, CH)])
        return 0

    lax.fori_loop(0, RPS // CH, zbody, 0)
    plsc.subcore_barrier()

    def body(j, _):
        row = base + j
        pltpu.sync_copy(dst_hbm.at[row], dA)
        if gather:
            pltpu.sync_copy(src_hbm.at[row], sA)
            pltpu.async_copy(tab_hbm.at[sA.at[0]], rows0, sem0).wait()
        else:
            pltpu.sync_copy(tab_hbm.at[row], rows0)
        pltpu.sync_copy(rows0, acc.at[dA.at[0]], add=True)
        return 0

    lax.fori_loop(0, cw, body, 0)
    plsc.subcore_barrier()

    pltpu.sync_copy(acc.at[pl.ds(sid * RPS, RPS)],
                    out_hbm.at[cid, pl.ds(sid * RPS, RPS)])


@functools.cache
def _sc_kernels():
    mesh = plsc.VectorSubcoreMesh(core_axis_name="c", subcore_axis_name="s",
                                  num_cores=NC, num_subcores=NS)

    scratches = [
        pltpu.VMEM((1, CH), jnp.int32),
        pltpu.VMEM((1, CH), jnp.int32),
        pltpu.VMEM((1, CH), jnp.int32),
        pltpu.VMEM((1, CH), jnp.int32),
        pltpu.VMEM((CH, F), jnp.float32),
        pltpu.VMEM((CH, F), jnp.float32),
        pltpu.VMEM_SHARED((NACC, F), jnp.float32),
        pltpu.SemaphoreType.DMA,
        pltpu.SemaphoreType.DMA,
    ]

    @functools.partial(
        pl.kernel,
        out_type=jax.ShapeDtypeStruct((NC, NACC, F), jnp.float32),
        mesh=mesh,
        scratch_types=scratches,
    )
    def _p_segsum(x_hbm, src3d, dst3d, out_hbm, sA, dA, sB, dB, rows0,
                  rows1, acc, sem0, sem1):
        _segsum_body(x_hbm, src3d, dst3d, out_hbm, sA, dA, sB, dB, rows0,
                     rows1, acc, sem0, sem1, gather=True)

    @functools.partial(
        pl.kernel,
        out_type=jax.ShapeDtypeStruct((NC, NACC, F), jnp.float32),
        mesh=mesh,
        scratch_types=scratches,
    )
    def _s_segsum(ea3d, src3d, dst3d, out_hbm, sA, dA, sB, dB, rows0,
                  rows1, acc, sem0, sem1):
        _segsum_body(ea3d, src3d, dst3d, out_hbm, sA, dA, sB, dB, rows0,
                     rows1, acc, sem0, sem1, gather=False)

    return _p_segsum, _s_segsum


def _embed_tc(h_ref, w_ref, b_ref, o_ref):
    o_ref[...] = jnp.dot(h_ref[...], w_ref[...],
                         preferred_element_type=jnp.float32) + b_ref[...]


def _layer_tc(x_ref, p_ref, s_ref, we_ref, wc_ref, bc_ref, g_ref, b_ref, o_ref):
    x = x_ref[...]
    s = s_ref[0, :N, :DE] + s_ref[1, :N, :DE]
    agg = p_ref[0, :N] + p_ref[1, :N] + jnp.dot(s, we_ref[...],
                                        preferred_element_type=jnp.float32)
    y = jnp.dot(x + agg, wc_ref[...],
                preferred_element_type=jnp.float32) + bc_ref[...]
    mu = jnp.mean(y, axis=0, keepdims=True)
    var = jnp.mean((y - mu) * (y - mu), axis=0, keepdims=True)
    y = (y - mu) * lax.rsqrt(var + BN_EPS)
    y = y * g_ref[...] + b_ref[...]
    o_ref[...] = jnp.maximum(y, 0.0) + x


def _pool_tc(x_ref, batch_ref, wout_ref, bout_ref, o_ref):
    x = x_ref[...]
    b = batch_ref[...]
    gids = lax.broadcasted_iota(jnp.int32, (G, 1), 0)
    onehot_t = (b == gids).astype(jnp.float32)
    sums = jnp.dot(onehot_t, x, preferred_element_type=jnp.float32)
    cnt = jnp.sum(onehot_t, axis=1, keepdims=True)
    pooled = sums / jnp.maximum(cnt, 1.0)
    o_ref[...] = jnp.dot(pooled, wout_ref[...],
                         preferred_element_type=jnp.float32) + bout_ref[...]


def kernel(h, edge_attr, edge_index, edge_feat_mat, pair_info, batch,
           W_embed, b_embed, Wconv, bconv, We, gamma, beta, Wout, bout):
    del edge_feat_mat, pair_info
    src = edge_index[0].astype(jnp.int32)
    dst = edge_index[1].astype(jnp.int32)
    pad = EPAD - E
    src3d = jnp.concatenate(
        [src, jnp.zeros((pad,), jnp.int32)]).reshape(NS * (CW0 + CW1), 1, CH)
    dst3d = jnp.concatenate(
        [dst, jnp.full((pad,), N, jnp.int32)]).reshape(NS * (CW0 + CW1), 1, CH)
    ea3d = jnp.pad(edge_attr, ((0, pad), (0, F - DE))).reshape(NS * (CW0 + CW1), CH, F)

    _p_segsum, _s_segsum = _sc_kernels()

    x = pl.pallas_call(
        _embed_tc,
        out_shape=jax.ShapeDtypeStruct((N, F), jnp.float32),
    )(h, W_embed, b_embed.reshape(1, F))

    s_part = _s_segsum(ea3d, src3d, dst3d)

    layer = pl.pallas_call(
        _layer_tc,
        out_shape=jax.ShapeDtypeStruct((N, F), jnp.float32),
    )
    for l in range(L):
        p_part = _p_segsum(x, src3d, dst3d)
        x = layer(x, p_part, s_part, We[l], Wconv[l], bconv[l].reshape(1, F),
                  gamma[l].reshape(1, F), beta[l].reshape(1, F))

    out = pl.pallas_call(
        _pool_tc,
        out_shape=jax.ShapeDtypeStruct((G, C), jnp.float32),
    )(x, batch.astype(jnp.int32).reshape(1, N), Wout, bout.reshape(1, C))
    return out


# speedup vs baseline: 1.2077x; 1.0078x over previous
import functools

import jax
import jax.numpy as jnp
from jax import lax
from jax.experimental import pallas as pl
from jax.experimental.pallas import tpu as pltpu
from jax.experimental.pallas import tpu_sc as plsc

N = 10000
E = 320000
F = 128
DE = 16
L = 4
C = 10
G = 64
BN_EPS = 1e-5

NC = 2
NS = 16
NW = NC * NS
CH = 128
CW0 = 104
CW1 = 54
EPAD = NS * CH * (CW0 + CW1)
NACC = 10240
RPS = NACC // NS


def _zero_vmem_rows(buf, rows, width):
    z16 = jnp.zeros((16,), jnp.float32)

    def body(i, _):
        for j in range(width // 16):
            buf[i, pl.ds(j * 16, 16)] = z16
        return 0

    lax.fori_loop(0, rows, body, 0)


def _segsum_body(tab_hbm, src_hbm, dst_hbm, out_hbm, sA, dA, sB, dB,
                 rows0, rows1, acc, sem0, sem1, *, gather):
    cid = lax.axis_index("c")
    sid = lax.axis_index("s")
    cw = jnp.where(cid == 0, CW0, CW1)
    base = jnp.where(cid == 0, sid * CW0, NS * CW0 + sid * CW1)

    _zero_vmem_rows(rows0, CH, F)

    def zbody(i, _):
        pltpu.sync_copy(rows0, acc.at[pl.ds(sid * RPS + i * CH, CH)])
        return 0

    lax.fori_loop(0, RPS // CH, zbody, 0)
    plsc.subcore_barrier()

    def body(j, _):
        row = base + j
        pltpu.sync_copy(dst_hbm.at[row], dA)
        if gather:
            pltpu.sync_copy(src_hbm.at[row], sA)
            pltpu.async_copy(tab_hbm.at[sA.at[0]], rows0, sem0).wait()
        else:
            pltpu.sync_copy(tab_hbm.at[row], rows0)
        pltpu.sync_copy(rows0, acc.at[dA.at[0]], add=True)
        return 0

    lax.fori_loop(0, cw, body, 0)
    plsc.subcore_barrier()

    pltpu.sync_copy(acc.at[pl.ds(sid * RPS, RPS)],
                    out_hbm.at[cid, pl.ds(sid * RPS, RPS)])


@functools.cache
def _sc_kernels():
    mesh = plsc.VectorSubcoreMesh(core_axis_name="c", subcore_axis_name="s",
                                  num_cores=NC, num_subcores=NS)

    scratches = [
        pltpu.VMEM((1, CH), jnp.int32),
        pltpu.VMEM((1, CH), jnp.int32),
        pltpu.VMEM((1, CH), jnp.int32),
        pltpu.VMEM((1, CH), jnp.int32),
        pltpu.VMEM((CH, F), jnp.float32),
        pltpu.VMEM((CH, F), jnp.float32),
        pltpu.VMEM_SHARED((NACC, F), jnp.float32),
        pltpu.SemaphoreType.DMA,
        pltpu.SemaphoreType.DMA,
    ]

    @functools.partial(
        pl.kernel,
        out_type=jax.ShapeDtypeStruct((NC, NACC, F), jnp.float32),
        mesh=mesh,
        scratch_types=scratches,
    )
    def _p_segsum(x_hbm, src3d, dst3d, out_hbm, sA, dA, sB, dB, rows0,
                  rows1, acc, sem0, sem1):
        _segsum_body(x_hbm, src3d, dst3d, out_hbm, sA, dA, sB, dB, rows0,
                     rows1, acc, sem0, sem1, gather=True)

    @functools.partial(
        pl.kernel,
        out_type=jax.ShapeDtypeStruct((NC, NACC, F), jnp.float32),
        mesh=mesh,
        scratch_types=scratches,
    )
    def _s_segsum(ea3d, src3d, dst3d, out_hbm, sA, dA, sB, dB, rows0,
                  rows1, acc, sem0, sem1):
        _segsum_body(ea3d, src3d, dst3d, out_hbm, sA, dA, sB, dB, rows0,
                     rows1, acc, sem0, sem1, gather=False)

    return _p_segsum, _s_segsum


def _embed_tc(h_ref, w_ref, b_ref, o_ref):
    o_ref[...] = jnp.dot(h_ref[...], w_ref[...],
                         preferred_element_type=jnp.float32) + b_ref[...]


def _layer_tc(x_ref, p_ref, s_ref, we_ref, wc_ref, bc_ref, g_ref, b_ref, o_ref):
    x = x_ref[...]
    s = s_ref[0, :N, :DE] + s_ref[1, :N, :DE]
    agg = p_ref[0, :N] + p_ref[1, :N] + jnp.dot(s, we_ref[...],
                                        preferred_element_type=jnp.float32)
    y = jnp.dot(x + agg, wc_ref[...],
                preferred_element_type=jnp.float32) + bc_ref[...]
    mu = jnp.mean(y, axis=0, keepdims=True)
    var = jnp.mean((y - mu) * (y - mu), axis=0, keepdims=True)
    y = (y - mu) * lax.rsqrt(var + BN_EPS)
    y = y * g_ref[...] + b_ref[...]
    o_ref[...] = jnp.maximum(y, 0.0) + x


def _pool_tc(x_ref, batch_ref, wout_ref, bout_ref, o_ref):
    x = x_ref[...]
    b = batch_ref[...]
    gids = lax.broadcasted_iota(jnp.int32, (G, 1), 0)
    onehot_t = (b == gids).astype(jnp.float32)
    sums = jnp.dot(onehot_t, x, preferred_element_type=jnp.float32)
    cnt = jnp.sum(onehot_t, axis=1, keepdims=True)
    pooled = sums / jnp.maximum(cnt, 1.0)
    o_ref[...] = jnp.dot(pooled, wout_ref[...],
                         preferred_element_type=jnp.float32) + bout_ref[...]


def kernel(h, edge_attr, edge_index, edge_feat_mat, pair_info, batch,
           W_embed, b_embed, Wconv, bconv, We, gamma, beta, Wout, bout):
    del edge_feat_mat, pair_info
    src = edge_index[0].astype(jnp.int32)
    dst = edge_index[1].astype(jnp.int32)
    pad = EPAD - E
    src3d = jnp.concatenate(
        [src, jnp.zeros((pad,), jnp.int32)]).reshape(NS * (CW0 + CW1), 1, CH)
    dst3d = jnp.concatenate(
        [dst, jnp.full((pad,), N, jnp.int32)]).reshape(NS * (CW0 + CW1), 1, CH)
    ea3d = jnp.pad(edge_attr, ((0, pad), (0, F - DE))).reshape(NS * (CW0 + CW1), CH, F)

    _p_segsum, _s_segsum = _sc_kernels()

    x = pl.pallas_call(
        _embed_tc,
        out_shape=jax.ShapeDtypeStruct((N, F), jnp.float32),
    )(h, W_embed, b_embed.reshape(1, F))

    s_part = _s_segsum(ea3d, src3d, dst3d)

    layer = pl.pallas_call(
        _layer_tc,
        out_shape=jax.ShapeDtypeStruct((N, F), jnp.float32),
    )
    for l in range(L):
        p_part = _p_segsum(x, src3d, dst3d)
        x = layer(x, p_part, s_part, We[l], Wconv[l], bconv[l].reshape(1, F),
                  gamma[l].reshape(1, F), beta[l].reshape(1, F))

    out = pl.pallas_call(
        _pool_tc,
        out_shape=jax.ShapeDtypeStruct((G, C), jnp.float32),
    )(x, batch.astype(jnp.int32).reshape(1, N), Wout, bout.reshape(1, C))
    return out
